# baseline (speedup 1.0000x reference)
import functools

import jax
import jax.numpy as jnp
from jax.experimental import pallas as pl
from jax.experimental.pallas import tpu as pltpu

C = 128
PAD = 16


def _wavenet_body(x_ref, wi_ref, wb_ref, wf_ref, bf_ref, out_ref,
                  wa, wb_s, *, L):
    ones = jnp.ones((L, C), jnp.bfloat16)
    wa[PAD:PAD + L, 3 * C:] = ones
    wb_s[PAD:PAD + L, 3 * C:] = ones

    def put(s, val, d):
        s[PAD + d:PAD + d + L, 0:C] = val
        s[PAD:PAD + L, C:2 * C] = val
        s[PAD - d:PAD - d + L, 2 * C:3 * C] = val
        z = jnp.zeros((d, C), jnp.bfloat16)
        s[PAD:PAD + d, 0:C] = z
        s[PAD + L - d:PAD + L, 2 * C:3 * C] = z

    half = jnp.bfloat16(0.5)
    one = jnp.bfloat16(1.0)

    def gate(hb):
        return half * jnp.tanh(hb) * (one + jnp.tanh(hb * half))

    put(wa, jnp.transpose(x_ref[0].astype(jnp.bfloat16)), 1)
    z = jnp.dot(wa[PAD:PAD + L, :], wi_ref[...],
                preferred_element_type=jnp.float32)
    x0 = z.astype(jnp.bfloat16)

    y2s = []
    for li in range(4):
        d = 1 << li
        s = wa if li % 2 == 0 else wb_s

        def block(idx):
            zz = jnp.dot(s[PAD:PAD + L, :], wb_ref[idx],
                         preferred_element_type=jnp.float32)
            zb = zz.astype(jnp.bfloat16)
            return gate(zb[:, :C]) + zb[:, C:]

        put(s, x0, d)
        y1 = block(li)
        put(s, y1, d)
        y2s.append(block(4 + li))

    ycat = jnp.concatenate(y2s, axis=1)
    acc = jnp.dot(ycat, wf_ref[...],
                  preferred_element_type=jnp.float32) + bf_ref[...]
    out_ref[0] = jnp.transpose(acc)


def _fuse_block(cw, cb, rw, rb):
    w = jnp.zeros((4 * C, 2 * C), jnp.float32)
    w = w.at[:3 * C, :C].set(cw.reshape(3 * C, C))
    w = w.at[C:2 * C, C:].set(rw)
    w = w.at[3 * C, :C].set(cb[0])
    w = w.at[3 * C, C:].set(rb[0])
    return w


def kernel(x, iw, ib, fw, fb,
           s0l0_cw, s0l0_cb, s0l0_rw, s0l0_rb,
           s0l1_cw, s0l1_cb, s0l1_rw, s0l1_rb,
           s0l2_cw, s0l2_cb, s0l2_rw, s0l2_rb,
           s0l3_cw, s0l3_cb, s0l3_rw, s0l3_rb,
           s1l0_cw, s1l0_cb, s1l0_rw, s1l0_rb, s1l0_sw, s1l0_sb,
           s1l1_cw, s1l1_cb, s1l1_rw, s1l1_rb, s1l1_sw, s1l1_sb,
           s1l2_cw, s1l2_cb, s1l2_rw, s1l2_rb, s1l2_sw, s1l2_sb,
           s1l3_cw, s1l3_cb, s1l3_rw, s1l3_rb, s1l3_sw, s1l3_sb):
    B, _, L = x.shape
    Lp = L + 2 * PAD

    blocks = [
        (s0l0_cw, s0l0_cb, s0l0_rw, s0l0_rb),
        (s0l1_cw, s0l1_cb, s0l1_rw, s0l1_rb),
        (s0l2_cw, s0l2_cb, s0l2_rw, s0l2_rb),
        (s0l3_cw, s0l3_cb, s0l3_rw, s0l3_rb),
        (s1l0_cw, s1l0_cb, s1l0_rw, s1l0_rb),
        (s1l1_cw, s1l1_cb, s1l1_rw, s1l1_rb),
        (s1l2_cw, s1l2_cb, s1l2_rw, s1l2_rb),
        (s1l3_cw, s1l3_cb, s1l3_rw, s1l3_rb),
    ]
    wb = jnp.stack([_fuse_block(*blk) for blk in blocks]
                   ).astype(jnp.bfloat16)

    wi = jnp.zeros((4 * C, C), jnp.float32)
    wi = wi.at[:3 * C, :].set(iw.reshape(3 * C, C))
    wi = wi.at[3 * C, :].set(ib[0])
    wi = wi.astype(jnp.bfloat16)

    wf_chain = jnp.concatenate([sw @ fw for sw in
                                (s1l0_sw, s1l1_sw, s1l2_sw, s1l3_sw)],
                               axis=0).astype(jnp.bfloat16)
    bsf = (s1l0_sb + s1l1_sb + s1l2_sb + s1l3_sb) @ fw + fb

    body = functools.partial(_wavenet_body, L=L)

    return pl.pallas_call(
        body,
        out_shape=jax.ShapeDtypeStruct((B, C, L), jnp.float32),
        grid=(B,),
        in_specs=[
            pl.BlockSpec((1, C, L), lambda b: (b, 0, 0)),
            pl.BlockSpec((4 * C, C), lambda b: (0, 0)),
            pl.BlockSpec((8, 4 * C, 2 * C), lambda b: (0, 0, 0)),
            pl.BlockSpec((4 * C, C), lambda b: (0, 0)),
            pl.BlockSpec((1, C), lambda b: (0, 0)),
        ],
        out_specs=pl.BlockSpec((1, C, L), lambda b: (b, 0, 0)),
        scratch_shapes=[
            pltpu.VMEM((Lp, 4 * C), jnp.bfloat16),
            pltpu.VMEM((Lp, 4 * C), jnp.bfloat16),
        ],
        compiler_params=pltpu.CompilerParams(
            dimension_semantics=("parallel",)),
    )(x, wi, wb, wf_chain, bsf)


# speedup vs baseline: 1.1488x; 1.1488x over previous
import functools

import jax
import jax.numpy as jnp
from jax.experimental import pallas as pl
from jax.experimental.pallas import tpu as pltpu

C = 128
PAD = 16


def _wavenet_body(x_ref, wi_ref, wb_ref, wf_ref, bf_ref, out_ref,
                  xs, x0s, y1a, y1b, y1c, y1d, *, L):
    y1s_all = (y1a, y1b, y1c, y1d)
    zh = jnp.zeros((PAD, C), jnp.bfloat16)
    for s in (xs, x0s) + y1s_all:
        s[0:PAD, :] = zh
        s[PAD + L:, :] = zh

    xs[PAD:PAD + L, :] = jnp.transpose(x_ref[0].astype(jnp.bfloat16))

    ones = jnp.ones((L, C), jnp.bfloat16)

    def taps(src, d):
        return jnp.concatenate(
            [src[PAD - d:PAD - d + L, :],
             src[PAD:PAD + L, :],
             src[PAD + d:PAD + d + L, :],
             ones], axis=1)

    z = jnp.dot(taps(xs, 1), wi_ref[...], preferred_element_type=jnp.float32)
    x0s[PAD:PAD + L, :] = z.astype(jnp.bfloat16)

    half = jnp.bfloat16(0.5)
    one = jnp.bfloat16(1.0)

    def gate(hb):
        return half * jnp.tanh(hb) * (one + jnp.tanh(hb * half))

    y2s = []
    for li in range(4):
        d = 1 << li
        y1s = y1s_all[li]

        def block(src, idx):
            zz = jnp.dot(taps(src, d), wb_ref[idx],
                         preferred_element_type=jnp.float32)
            zb = zz.astype(jnp.bfloat16)
            return gate(zb[:, :C]) + zb[:, C:]

        y1s[PAD:PAD + L, :] = block(x0s, li)
        y2s.append(block(y1s, 4 + li))

    ycat = jnp.concatenate(y2s, axis=1)
    acc = jnp.dot(ycat, wf_ref[...],
                  preferred_element_type=jnp.float32) + bf_ref[...]
    out_ref[0] = jnp.transpose(acc)


def _fuse_block(cw, cb, rw, rb):
    w = jnp.zeros((4 * C, 2 * C), jnp.float32)
    w = w.at[:3 * C, :C].set(cw.reshape(3 * C, C))
    w = w.at[C:2 * C, C:].set(rw)
    w = w.at[3 * C, :C].set(cb[0])
    w = w.at[3 * C, C:].set(rb[0])
    return w


def kernel(x, iw, ib, fw, fb,
           s0l0_cw, s0l0_cb, s0l0_rw, s0l0_rb,
           s0l1_cw, s0l1_cb, s0l1_rw, s0l1_rb,
           s0l2_cw, s0l2_cb, s0l2_rw, s0l2_rb,
           s0l3_cw, s0l3_cb, s0l3_rw, s0l3_rb,
           s1l0_cw, s1l0_cb, s1l0_rw, s1l0_rb, s1l0_sw, s1l0_sb,
           s1l1_cw, s1l1_cb, s1l1_rw, s1l1_rb, s1l1_sw, s1l1_sb,
           s1l2_cw, s1l2_cb, s1l2_rw, s1l2_rb, s1l2_sw, s1l2_sb,
           s1l3_cw, s1l3_cb, s1l3_rw, s1l3_rb, s1l3_sw, s1l3_sb):
    B, _, L = x.shape
    Lp = L + 2 * PAD

    blocks = [
        (s0l0_cw, s0l0_cb, s0l0_rw, s0l0_rb),
        (s0l1_cw, s0l1_cb, s0l1_rw, s0l1_rb),
        (s0l2_cw, s0l2_cb, s0l2_rw, s0l2_rb),
        (s0l3_cw, s0l3_cb, s0l3_rw, s0l3_rb),
        (s1l0_cw, s1l0_cb, s1l0_rw, s1l0_rb),
        (s1l1_cw, s1l1_cb, s1l1_rw, s1l1_rb),
        (s1l2_cw, s1l2_cb, s1l2_rw, s1l2_rb),
        (s1l3_cw, s1l3_cb, s1l3_rw, s1l3_rb),
    ]
    wb = jnp.stack([_fuse_block(*blk) for blk in blocks]
                   ).astype(jnp.bfloat16)

    wi = jnp.zeros((4 * C, C), jnp.float32)
    wi = wi.at[:3 * C, :].set(iw.reshape(3 * C, C))
    wi = wi.at[3 * C, :].set(ib[0])
    wi = wi.astype(jnp.bfloat16)

    wf_chain = jnp.concatenate([sw @ fw for sw in
                                (s1l0_sw, s1l1_sw, s1l2_sw, s1l3_sw)],
                               axis=0).astype(jnp.bfloat16)
    bsf = (s1l0_sb + s1l1_sb + s1l2_sb + s1l3_sb) @ fw + fb

    body = functools.partial(_wavenet_body, L=L)

    return pl.pallas_call(
        body,
        out_shape=jax.ShapeDtypeStruct((B, C, L), jnp.float32),
        grid=(B,),
        in_specs=[
            pl.BlockSpec((1, C, L), lambda b: (b, 0, 0)),
            pl.BlockSpec((4 * C, C), lambda b: (0, 0)),
            pl.BlockSpec((8, 4 * C, 2 * C), lambda b: (0, 0, 0)),
            pl.BlockSpec((4 * C, C), lambda b: (0, 0)),
            pl.BlockSpec((1, C), lambda b: (0, 0)),
        ],
        out_specs=pl.BlockSpec((1, C, L), lambda b: (b, 0, 0)),
        scratch_shapes=[
            pltpu.VMEM((Lp, C), jnp.bfloat16),
            pltpu.VMEM((Lp, C), jnp.bfloat16),
            pltpu.VMEM((Lp, C), jnp.bfloat16),
            pltpu.VMEM((Lp, C), jnp.bfloat16),
            pltpu.VMEM((Lp, C), jnp.bfloat16),
            pltpu.VMEM((Lp, C), jnp.bfloat16),
        ],
        compiler_params=pltpu.CompilerParams(
            dimension_semantics=("parallel",)),
    )(x, wi, wb, wf_chain, bsf)


# speedup vs baseline: 1.1677x; 1.0164x over previous
import functools

import jax
import jax.numpy as jnp
from jax.experimental import pallas as pl
from jax.experimental.pallas import tpu as pltpu

C = 128
PAD = 16


def _wavenet_body(x_ref, wi_ref, wb_ref, wf_ref, bf_ref, out_ref,
                  xs, x0s, *, L):
    zh = jnp.zeros((PAD, C), jnp.bfloat16)
    for s in (xs, x0s):
        s[0:PAD, :] = zh
        s[PAD + L:, :] = zh

    xs[PAD:PAD + L, :] = jnp.transpose(x_ref[0].astype(jnp.bfloat16))

    ones = jnp.ones((L, C), jnp.bfloat16)

    def taps(src, d):
        return jnp.concatenate(
            [src[PAD - d:PAD - d + L, :],
             src[PAD:PAD + L, :],
             src[PAD + d:PAD + d + L, :],
             ones], axis=1)

    z = jnp.dot(taps(xs, 1), wi_ref[...], preferred_element_type=jnp.float32)
    x0s[PAD:PAD + L, :] = z.astype(jnp.bfloat16)

    half = jnp.bfloat16(0.5)
    one = jnp.bfloat16(1.0)

    def gate(hb):
        return half * jnp.tanh(hb) * (one + jnp.tanh(hb * half))

    y2s = []
    for li in range(4):
        d = 1 << li

        def block(lhs, idx):
            zz = jnp.dot(lhs, wb_ref[idx],
                         preferred_element_type=jnp.float32)
            zb = zz.astype(jnp.bfloat16)
            return gate(zb[:, :C]) + zb[:, C:]

        y1 = block(taps(x0s, d), li)
        y1p = jnp.pad(y1, ((d, d), (0, 0)))
        xc1 = jnp.concatenate(
            [y1p[0:L, :], y1p[d:d + L, :], y1p[2 * d:2 * d + L, :], ones],
            axis=1)
        y2s.append(block(xc1, 4 + li))

    ycat = jnp.concatenate(y2s, axis=1)
    acc = jnp.dot(ycat, wf_ref[...],
                  preferred_element_type=jnp.float32) + bf_ref[...]
    out_ref[0] = jnp.transpose(acc)


def _fuse_block(cw, cb, rw, rb):
    w = jnp.zeros((4 * C, 2 * C), jnp.float32)
    w = w.at[:3 * C, :C].set(cw.reshape(3 * C, C))
    w = w.at[C:2 * C, C:].set(rw)
    w = w.at[3 * C, :C].set(cb[0])
    w = w.at[3 * C, C:].set(rb[0])
    return w


def kernel(x, iw, ib, fw, fb,
           s0l0_cw, s0l0_cb, s0l0_rw, s0l0_rb,
           s0l1_cw, s0l1_cb, s0l1_rw, s0l1_rb,
           s0l2_cw, s0l2_cb, s0l2_rw, s0l2_rb,
           s0l3_cw, s0l3_cb, s0l3_rw, s0l3_rb,
           s1l0_cw, s1l0_cb, s1l0_rw, s1l0_rb, s1l0_sw, s1l0_sb,
           s1l1_cw, s1l1_cb, s1l1_rw, s1l1_rb, s1l1_sw, s1l1_sb,
           s1l2_cw, s1l2_cb, s1l2_rw, s1l2_rb, s1l2_sw, s1l2_sb,
           s1l3_cw, s1l3_cb, s1l3_rw, s1l3_rb, s1l3_sw, s1l3_sb):
    B, _, L = x.shape
    Lp = L + 2 * PAD

    blocks = [
        (s0l0_cw, s0l0_cb, s0l0_rw, s0l0_rb),
        (s0l1_cw, s0l1_cb, s0l1_rw, s0l1_rb),
        (s0l2_cw, s0l2_cb, s0l2_rw, s0l2_rb),
        (s0l3_cw, s0l3_cb, s0l3_rw, s0l3_rb),
        (s1l0_cw, s1l0_cb, s1l0_rw, s1l0_rb),
        (s1l1_cw, s1l1_cb, s1l1_rw, s1l1_rb),
        (s1l2_cw, s1l2_cb, s1l2_rw, s1l2_rb),
        (s1l3_cw, s1l3_cb, s1l3_rw, s1l3_rb),
    ]
    wb = jnp.stack([_fuse_block(*blk) for blk in blocks]
                   ).astype(jnp.bfloat16)

    wi = jnp.zeros((4 * C, C), jnp.float32)
    wi = wi.at[:3 * C, :].set(iw.reshape(3 * C, C))
    wi = wi.at[3 * C, :].set(ib[0])
    wi = wi.astype(jnp.bfloat16)

    wf_chain = jnp.concatenate([sw @ fw for sw in
                                (s1l0_sw, s1l1_sw, s1l2_sw, s1l3_sw)],
                               axis=0).astype(jnp.bfloat16)
    bsf = (s1l0_sb + s1l1_sb + s1l2_sb + s1l3_sb) @ fw + fb

    body = functools.partial(_wavenet_body, L=L)

    return pl.pallas_call(
        body,
        out_shape=jax.ShapeDtypeStruct((B, C, L), jnp.float32),
        grid=(B,),
        in_specs=[
            pl.BlockSpec((1, C, L), lambda b: (b, 0, 0)),
            pl.BlockSpec((4 * C, C), lambda b: (0, 0)),
            pl.BlockSpec((8, 4 * C, 2 * C), lambda b: (0, 0, 0)),
            pl.BlockSpec((4 * C, C), lambda b: (0, 0)),
            pl.BlockSpec((1, C), lambda b: (0, 0)),
        ],
        out_specs=pl.BlockSpec((1, C, L), lambda b: (b, 0, 0)),
        scratch_shapes=[
            pltpu.VMEM((Lp, C), jnp.bfloat16),
            pltpu.VMEM((Lp, C), jnp.bfloat16),
        ],
        compiler_params=pltpu.CompilerParams(
            dimension_semantics=("parallel",)),
    )(x, wi, wb, wf_chain, bsf)


# speedup vs baseline: 1.2157x; 1.0412x over previous
import functools

import jax
import jax.numpy as jnp
from jax.experimental import pallas as pl
from jax.experimental.pallas import tpu as pltpu

C = 128


def _wavenet_body(x_ref, wi_ref, wb_ref, wf_ref, bf_ref, out_ref, *, L):
    ones = jnp.ones((L, C), jnp.bfloat16)

    def taps(val, d):
        vp = jnp.pad(val, ((d, d), (0, 0)))
        return jnp.concatenate(
            [vp[0:L, :], vp[d:d + L, :], vp[2 * d:2 * d + L, :], ones],
            axis=1)

    xt = jnp.transpose(x_ref[0].astype(jnp.bfloat16))

    z = jnp.dot(taps(xt, 1), wi_ref[...], preferred_element_type=jnp.float32)
    x0 = z.astype(jnp.bfloat16)

    half = jnp.bfloat16(0.5)
    one = jnp.bfloat16(1.0)

    def gate(hb):
        return half * jnp.tanh(hb) * (one + jnp.tanh(hb * half))

    y2s = []
    for li in range(4):
        d = 1 << li

        def block(lhs, idx):
            zz = jnp.dot(lhs, wb_ref[idx],
                         preferred_element_type=jnp.float32)
            zb = zz.astype(jnp.bfloat16)
            return gate(zb[:, :C]) + zb[:, C:]

        y1 = block(taps(x0, d), li)
        y2s.append(block(taps(y1, d), 4 + li))

    ycat = jnp.concatenate(y2s, axis=1)
    acc = jnp.dot(ycat, wf_ref[...],
                  preferred_element_type=jnp.float32) + bf_ref[...]
    out_ref[0] = jnp.transpose(acc)


def _fuse_block(cw, cb, rw, rb):
    w = jnp.zeros((4 * C, 2 * C), jnp.float32)
    w = w.at[:3 * C, :C].set(cw.reshape(3 * C, C))
    w = w.at[C:2 * C, C:].set(rw)
    w = w.at[3 * C, :C].set(cb[0])
    w = w.at[3 * C, C:].set(rb[0])
    return w


def kernel(x, iw, ib, fw, fb,
           s0l0_cw, s0l0_cb, s0l0_rw, s0l0_rb,
           s0l1_cw, s0l1_cb, s0l1_rw, s0l1_rb,
           s0l2_cw, s0l2_cb, s0l2_rw, s0l2_rb,
           s0l3_cw, s0l3_cb, s0l3_rw, s0l3_rb,
           s1l0_cw, s1l0_cb, s1l0_rw, s1l0_rb, s1l0_sw, s1l0_sb,
           s1l1_cw, s1l1_cb, s1l1_rw, s1l1_rb, s1l1_sw, s1l1_sb,
           s1l2_cw, s1l2_cb, s1l2_rw, s1l2_rb, s1l2_sw, s1l2_sb,
           s1l3_cw, s1l3_cb, s1l3_rw, s1l3_rb, s1l3_sw, s1l3_sb):
    B, _, L = x.shape

    blocks = [
        (s0l0_cw, s0l0_cb, s0l0_rw, s0l0_rb),
        (s0l1_cw, s0l1_cb, s0l1_rw, s0l1_rb),
        (s0l2_cw, s0l2_cb, s0l2_rw, s0l2_rb),
        (s0l3_cw, s0l3_cb, s0l3_rw, s0l3_rb),
        (s1l0_cw, s1l0_cb, s1l0_rw, s1l0_rb),
        (s1l1_cw, s1l1_cb, s1l1_rw, s1l1_rb),
        (s1l2_cw, s1l2_cb, s1l2_rw, s1l2_rb),
        (s1l3_cw, s1l3_cb, s1l3_rw, s1l3_rb),
    ]
    wb = jnp.stack([_fuse_block(*blk) for blk in blocks]
                   ).astype(jnp.bfloat16)

    wi = jnp.zeros((4 * C, C), jnp.float32)
    wi = wi.at[:3 * C, :].set(iw.reshape(3 * C, C))
    wi = wi.at[3 * C, :].set(ib[0])
    wi = wi.astype(jnp.bfloat16)

    wf_chain = jnp.concatenate([sw @ fw for sw in
                                (s1l0_sw, s1l1_sw, s1l2_sw, s1l3_sw)],
                               axis=0).astype(jnp.bfloat16)
    bsf = (s1l0_sb + s1l1_sb + s1l2_sb + s1l3_sb) @ fw + fb

    body = functools.partial(_wavenet_body, L=L)

    return pl.pallas_call(
        body,
        out_shape=jax.ShapeDtypeStruct((B, C, L), jnp.float32),
        grid=(B,),
        in_specs=[
            pl.BlockSpec((1, C, L), lambda b: (b, 0, 0)),
            pl.BlockSpec((4 * C, C), lambda b: (0, 0)),
            pl.BlockSpec((8, 4 * C, 2 * C), lambda b: (0, 0, 0)),
            pl.BlockSpec((4 * C, C), lambda b: (0, 0)),
            pl.BlockSpec((1, C), lambda b: (0, 0)),
        ],
        out_specs=pl.BlockSpec((1, C, L), lambda b: (b, 0, 0)),
        compiler_params=pltpu.CompilerParams(
            dimension_semantics=("parallel",)),
    )(x, wi, wb, wf_chain, bsf)
